# baseline (speedup 1.0000x reference)
import functools

import jax
import jax.numpy as jnp
from jax.experimental import pallas as pl
from jax.experimental.pallas import tpu as pltpu

N = 10000
E = 160000
D = 32
NG = 256
STEPS = 6

EBLK = 512


def _leaky(x):
    return jnp.where(x >= 0, x, 0.01 * x)


def _matvec_body(xj_ref, ew_ref, o_ref):
    xj = xj_ref[...]
    ew = ew_ref[...].astype(jnp.float32)
    acc = xj[:, 0:1] * ew[:, 0:D]
    for i in range(1, D):
        acc += xj[:, i:i + 1] * ew[:, i * D:(i + 1) * D]
    o_ref[...] = acc


def _edge_matvec(xj, ew):
    grid = E // EBLK
    return pl.pallas_call(
        _matvec_body,
        grid=(grid,),
        in_specs=[
            pl.BlockSpec((EBLK, D), lambda i: (i, 0)),
            pl.BlockSpec((EBLK, D * D), lambda i: (i, 0)),
        ],
        out_specs=pl.BlockSpec((EBLK, D), lambda i: (i, 0)),
        out_shape=jax.ShapeDtypeStruct((E, D), jnp.float32),
    )(xj, ew)


def _gru(m, h, Wih, Whh, bih, bhh):
    gi = m @ Wih.T + bih
    gh = h @ Whh.T + bhh
    ir, iz, inn = jnp.split(gi, 3, axis=-1)
    hr, hz, hn = jnp.split(gh, 3, axis=-1)
    r = jax.nn.sigmoid(ir + hr)
    z = jax.nn.sigmoid(iz + hz)
    n = jnp.tanh(inn + r * hn)
    return (1.0 - z) * n + z * h


def kernel(x, edge_index, edge_attr, batch, stem_atmidx, jbond_atmidx, W0, b0, We1, be1, We2, be2, Wroot, cbias, Wih, Whh, bih, bhh, W1, b1, lWih, lWhh, lbih, lbhh, W3, b3, Ws1, bs1, Ws2, bs2, Wj1, bj1, Wj2, bj2):
    src, dst = edge_index[0], edge_index[1]
    out = _leaky(x @ W0.T + b0)
    h = out
    ew = _leaky(edge_attr @ We1.T + be1)
    ew = (ew @ We2.T + be2).astype(jnp.bfloat16)
    deg = jax.ops.segment_sum(jnp.ones((E,), jnp.float32), dst, num_segments=N)
    deg = jnp.maximum(deg, 1.0)[:, None]
    for _ in range(STEPS):
        xj = out[src]
        msg = _edge_matvec(xj, ew)
        agg = jax.ops.segment_sum(msg, dst, num_segments=N) / deg
        m = _leaky(agg + out @ Wroot.T + cbias)
        out = _gru(m, h, Wih, Whh, bih, bhh)
        h = out
    per_atom = _leaky(out @ W1.T + b1)
    stem_preds = _leaky(per_atom[stem_atmidx] @ Ws1.T + bs1) @ Ws2.T + bs2
    jg = per_atom[jbond_atmidx.reshape(-1)]
    jbond_preds = (_leaky(jg @ Wj1.T + bj1) @ Wj2.T + bj2).reshape(jbond_atmidx.shape).mean(axis=1)
    qstar = jnp.zeros((NG, 2 * D), jnp.float32)
    hh = jnp.zeros((NG, D), jnp.float32)
    cc = jnp.zeros((NG, D), jnp.float32)
    for _ in range(3):
        g = qstar @ lWih.T + lbih + hh @ lWhh.T + lbhh
        i, f, gg, o = jnp.split(g, 4, axis=-1)
        i = jax.nn.sigmoid(i)
        f = jax.nn.sigmoid(f)
        o = jax.nn.sigmoid(o)
        gg = jnp.tanh(gg)
        cc = f * cc + i * gg
        hh = o * jnp.tanh(cc)
        q = hh
        e = jnp.sum(out * q[batch], axis=-1)
        emax = jax.ops.segment_max(e, batch, num_segments=NG)
        ee = jnp.exp(e - emax[batch])
        denom = jax.ops.segment_sum(ee, batch, num_segments=NG)
        a = ee / (denom[batch] + 1e-16)
        r = jax.ops.segment_sum(a[:, None] * out, batch, num_segments=NG)
        qstar = jnp.concatenate([q, r], axis=-1)
    sout = qstar @ W3.T + b3
    return sout, stem_preds, jbond_preds


# speedup vs baseline: 3.0992x; 3.0992x over previous
import functools

import jax
import jax.numpy as jnp
from jax import lax
from jax.experimental import pallas as pl
from jax.experimental.pallas import tpu as pltpu
from jax.experimental.pallas import tpu_sc as plsc

N = 10000
NP = 10240
E = 160000
EP = 163840
D = 32
NG = 256
STEPS = 6

NW = 32
CH = 128
ECHUNKS = EP // (NW * CH)
EBLK = 1024
KA = 144
RPT = NP // 16

_SC_MESH = dict(core_axis_name="c", subcore_axis_name="s")


def _leaky(x):
    return jnp.where(x >= 0, x, 0.01 * x)


def _make_gather(nchunks):
    rows = NW * nchunks * CH

    @functools.partial(
        pl.kernel,
        out_type=jax.ShapeDtypeStruct((rows, 128), jnp.float32),
        mesh=plsc.VectorSubcoreMesh(**_SC_MESH),
        scratch_types=[
            pltpu.VMEM((nchunks, CH), jnp.int32),
            pltpu.VMEM((CH, 128), jnp.float32),
            pltpu.VMEM((CH, 128), jnp.float32),
            pltpu.SemaphoreType.DMA,
            pltpu.SemaphoreType.DMA,
        ],
    )
    def gather_k(table_hbm, idx_hbm, out_hbm, idx_v, buf0, buf1, sem0, sem1):
        c = lax.axis_index("c")
        s = lax.axis_index("s")
        wid = s * 2 + c
        pltpu.sync_copy(idx_hbm.at[wid], idx_v)
        bufs = (buf0, buf1)
        sems = (sem0, sem1)
        pltpu.async_copy(table_hbm.at[idx_v.at[0]], bufs[0], sems[0])

        def body(t, _):
            for b in range(2):
                j = 2 * t + b

                @pl.when(j + 1 < nchunks)
                def _():
                    pltpu.async_copy(
                        table_hbm.at[idx_v.at[j + 1]], bufs[(b + 1) % 2],
                        sems[(b + 1) % 2])
                cur = bufs[b]
                pltpu.make_async_copy(
                    table_hbm.at[idx_v.at[j]], cur, sems[b]).wait()
                pltpu.sync_copy(
                    cur, out_hbm.at[pl.ds(wid * nchunks * CH + j * CH, CH)])
            return ()

        lax.fori_loop(0, nchunks // 2, body, (), unroll=False)

    return gather_k


_gather_main = _make_gather(ECHUNKS)
_gather_epi = _make_gather(2)


@functools.partial(
    pl.kernel,
    out_type=jax.ShapeDtypeStruct((2, NP, 128), jnp.float32),
    mesh=plsc.VectorSubcoreMesh(**_SC_MESH),
    scratch_types=[
        pltpu.VMEM((ECHUNKS, CH), jnp.int32),
        pltpu.VMEM((CH, 128), jnp.float32),
        pltpu.VMEM((CH, 128), jnp.float32),
        pltpu.VMEM_SHARED((NP, 128), jnp.float32),
        pltpu.SemaphoreType.DMA,
        pltpu.SemaphoreType.DMA,
    ],
)
def _scatter_k(msg_hbm, idx_hbm, zrow_hbm, out_hbm,
               idx_v, buf0, buf1, acc, sem0, sem1):
    c = lax.axis_index("c")
    s = lax.axis_index("s")
    wid = s * 2 + c
    pltpu.sync_copy(zrow_hbm.at[pl.ds(s * RPT, RPT)],
                    acc.at[pl.ds(s * RPT, RPT)])
    pltpu.sync_copy(idx_hbm.at[wid], idx_v)
    plsc.subcore_barrier()
    bufs = (buf0, buf1)
    sems = (sem0, sem1)
    base = wid * ECHUNKS * CH
    pltpu.async_copy(msg_hbm.at[pl.ds(base, CH)], buf0, sems[0])

    def body(t, _):
        for b in range(2):
            j = 2 * t + b

            @pl.when(j + 1 < ECHUNKS)
            def _():
                pltpu.async_copy(msg_hbm.at[pl.ds(base + (j + 1) * CH, CH)],
                                 bufs[(b + 1) % 2], sems[(b + 1) % 2])
            cur = bufs[b]
            pltpu.make_async_copy(msg_hbm.at[pl.ds(base + j * CH, CH)],
                                  cur, sems[b]).wait()
            pltpu.sync_copy(cur, acc.at[idx_v.at[j]], add=True)
        return ()

    lax.fori_loop(0, ECHUNKS // 2, body, (), unroll=False)
    plsc.subcore_barrier()
    pltpu.sync_copy(acc.at[pl.ds(s * RPT, RPT)],
                    out_hbm.at[c, pl.ds(s * RPT, RPT)])


def _edge_body(h_ref, xj_ref, w_ref, o_ref):
    h = h_ref[...]
    xj = xj_ref[:, 0:D]
    w = w_ref[...]
    ewT = lax.dot_general(w, h, (((1,), (0,)), ((), ())),
                          preferred_element_type=jnp.float32)
    ident = (lax.broadcasted_iota(jnp.int32, (D, D), 0)
             == lax.broadcasted_iota(jnp.int32, (D, D), 1)).astype(jnp.float32)
    xjT = lax.dot_general(ident, xj, (((1,), (1,)), ((), ())),
                          preferred_element_type=jnp.float32)
    acc = jnp.zeros((D, EBLK), jnp.float32)
    for i in range(D):
        xi = jnp.broadcast_to(xjT[i:i + 1, :], (D, EBLK))
        acc = acc + xi * ewT[i * D:(i + 1) * D, :]
    msg = lax.dot_general(acc, ident, (((0,), (0,)), ((), ())),
                          preferred_element_type=jnp.float32)
    o_ref[...] = jnp.concatenate(
        [msg, jnp.zeros((EBLK, 128 - D), jnp.float32)], axis=1)


def _edge_matvec(h128aT, xj, We2aug):
    grid = EP // EBLK
    return pl.pallas_call(
        _edge_body,
        grid=(grid,),
        in_specs=[
            pl.BlockSpec((KA, EBLK), lambda i: (0, i)),
            pl.BlockSpec((EBLK, 128), lambda i: (i, 0)),
            pl.BlockSpec((D * D, KA), lambda i: (0, 0)),
        ],
        out_specs=pl.BlockSpec((EBLK, 128), lambda i: (i, 0)),
        out_shape=jax.ShapeDtypeStruct((EP, 128), jnp.float32),
    )(h128aT, xj, We2aug)


def kernel(x, edge_index, edge_attr, batch, stem_atmidx, jbond_atmidx, W0, b0, We1, be1, We2, be2, Wroot, cbias, Wih, Whh, bih, bhh, W1, b1, lWih, lWhh, lbih, lbhh, W3, b3, Ws1, bs1, Ws2, bs2, Wj1, bj1, Wj2, bj2):
    src = jnp.pad(edge_index[0], (0, EP - E))
    dst = jnp.pad(edge_index[1], (0, EP - E),
                  constant_values=NP - 1)
    src_l = src.reshape(NW, ECHUNKS, CH)
    dst_l = dst.reshape(NW, ECHUNKS, CH)

    xP = jnp.pad(x, ((0, NP - N), (0, 0)))
    out = _leaky(xP @ W0.T + b0)
    h = out

    eaP = jnp.pad(edge_attr, ((0, EP - E), (0, 0)))
    h128 = _leaky(eaP @ We1.T + be1)
    h128aT = jnp.concatenate([
        h128.T.astype(jnp.bfloat16),
        jnp.ones((1, EP), jnp.bfloat16),
        jnp.zeros((KA - 129, EP), jnp.bfloat16),
    ], axis=0)
    We2aug = jnp.concatenate([
        We2.astype(jnp.bfloat16),
        be2[:, None].astype(jnp.bfloat16),
        jnp.zeros((D * D, KA - 129), jnp.bfloat16),
    ], axis=1)

    zrow = jnp.zeros((NP, 128), jnp.float32)
    onesE = jnp.ones((EP, 128), jnp.float32)
    deg2 = _scatter_k(onesE, dst_l, zrow)
    deg = jnp.maximum(deg2[0, :, 0] + deg2[1, :, 0], 1.0)[:, None]

    for _ in range(STEPS):
        out128 = jnp.pad(out, ((0, 0), (0, 128 - D)))
        xj = _gather_main(out128, src_l)
        msg = _edge_matvec(h128aT, xj, We2aug)
        agg2 = _scatter_k(msg, dst_l, zrow)
        agg = agg2[0, :, :D] + agg2[1, :, :D]
        agg = agg / deg
        m = _leaky(agg + out @ Wroot.T + cbias)
        gi = m @ Wih.T + bih
        gh = h @ Whh.T + bhh
        ir, iz, inn = jnp.split(gi, 3, axis=-1)
        hr, hz, hn = jnp.split(gh, 3, axis=-1)
        r = jax.nn.sigmoid(ir + hr)
        z = jax.nn.sigmoid(iz + hz)
        n = jnp.tanh(inn + r * hn)
        out = (1.0 - z) * n + z * h
        h = out

    idx_epi = jnp.concatenate([
        stem_atmidx, jnp.zeros((48,), jnp.int32),
        jbond_atmidx[:, 0], jnp.zeros((4,), jnp.int32),
        jbond_atmidx[:, 1], jnp.zeros((4,), jnp.int32),
        jnp.zeros((8192 - 5056,), jnp.int32),
    ])
    out128 = jnp.pad(out, ((0, 0), (0, 128 - D)))
    g_epi = _gather_epi(out128, idx_epi.reshape(NW, 2, CH))[:, :D]
    pa_s = _leaky(g_epi[:2048] @ W1.T + b1)
    stem_preds = (_leaky(pa_s @ Ws1.T + bs1) @ Ws2.T + bs2)[:2000]
    pa_j = _leaky(g_epi[2048:5056] @ W1.T + b1)
    vj = (_leaky(pa_j @ Wj1.T + bj1) @ Wj2.T + bj2)
    jbond_preds = (0.5 * (vj[:1504] + vj[1504:]))[:1500, 0]

    outN = out[:N]
    qstar = jnp.zeros((NG, 2 * D), jnp.float32)
    hh = jnp.zeros((NG, D), jnp.float32)
    cc = jnp.zeros((NG, D), jnp.float32)
    for _ in range(3):
        g = qstar @ lWih.T + lbih + hh @ lWhh.T + lbhh
        i, f, gg, o = jnp.split(g, 4, axis=-1)
        i = jax.nn.sigmoid(i)
        f = jax.nn.sigmoid(f)
        o = jax.nn.sigmoid(o)
        gg = jnp.tanh(gg)
        cc = f * cc + i * gg
        hh = o * jnp.tanh(cc)
        q = hh
        e = jnp.sum(outN * q[batch], axis=-1)
        emax = jax.ops.segment_max(e, batch, num_segments=NG)
        ee = jnp.exp(e - emax[batch])
        denom = jax.ops.segment_sum(ee, batch, num_segments=NG)
        a = ee / (denom[batch] + 1e-16)
        r = jax.ops.segment_sum(a[:, None] * outN, batch, num_segments=NG)
        qstar = jnp.concatenate([q, r], axis=-1)
    sout = qstar @ W3.T + b3
    return sout, stem_preds, jbond_preds
